# baseline (speedup 1.0000x reference)
import functools

import jax
import jax.numpy as jnp
from jax import lax
from jax.experimental import pallas as pl
from jax.experimental.pallas import tpu as pltpu
from jax.experimental.pallas import tpu_sc as plsc

_NC = 2
_NS = 16
_NW = _NC * _NS
_G = 128
_LPC = 2
_NBUF = 2


@functools.lru_cache(maxsize=None)
def _build(B, L, dim):
    b_per_w = B // _NW
    chunks_per_w = L // _LPC
    steps = chunks_per_w // _NBUF

    mesh = plsc.VectorSubcoreMesh(core_axis_name="c", subcore_axis_name="s")

    @functools.partial(
        pl.kernel,
        mesh=mesh,
        out_type=jax.ShapeDtypeStruct((L, dim, B), jnp.float32),
        compiler_params=pltpu.CompilerParams(use_tc_tiling_on_sc=False, needs_layout_passes=False),
        scratch_types=[
            pltpu.VMEM((L, _G), jnp.int32),
            pltpu.VMEM((_NBUF, _LPC, _G, dim), jnp.float32),
            pltpu.VMEM((_NBUF, _LPC, dim, _G), jnp.float32),
            pltpu.SemaphoreType.DMA,
            pltpu.SemaphoreType.DMA,
            pltpu.SemaphoreType.DMA,
            pltpu.SemaphoreType.DMA,
        ],
    )
    def gather_kernel(idx_hbm, table_hbm, out_hbm,
                      idx_v, rows, outb, gsem0, gsem1, wsem0, wsem1):
        gsems = [gsem0, gsem1]
        wsems = [wsem0, wsem1]
        wid = lax.axis_index("s") * _NC + lax.axis_index("c")
        b_base = wid * b_per_w

        pltpu.sync_copy(idx_hbm.at[wid], idx_v)

        lanes = lax.iota(jnp.int32, 16)

        def step(i, carry):
            for nb in range(_NBUF):
                c = i * _NBUF + nb
                l0 = c * _LPC

                @pl.when(i >= 1)
                def _drain():
                    pltpu.make_async_copy(
                        outb.at[nb],
                        out_hbm.at[pl.ds(0, _LPC), :, pl.ds(0, _G)],
                        wsems[nb],
                    ).wait()

                copies = []
                for j in range(_LPC):
                    copies.append(pltpu.make_async_copy(
                        table_hbm.at[idx_v.at[l0 + j]],
                        rows.at[nb, j],
                        gsems[nb],
                    ))
                for cp in copies:
                    cp.start()
                for cp in copies:
                    cp.wait()

                for j in range(_LPC):
                    def tbody(d, carry):
                        dvec = jnp.full((16,), d, jnp.int32)
                        for kk in range(_G // 16):
                            bvec = lanes + 16 * kk
                            v = plsc.load_gather(rows.at[nb, j], [bvec, dvec])
                            plsc.store_scatter(outb.at[nb, j], [dvec, bvec], v)
                        return carry
                    lax.fori_loop(0, dim, tbody, 0)

                pltpu.make_async_copy(
                    outb.at[nb],
                    out_hbm.at[pl.ds(l0, _LPC), :, pl.ds(b_base, _G)],
                    wsems[nb],
                ).start()
            return carry

        lax.fori_loop(0, steps, step, 0)

        for nb in range(_NBUF):
            pltpu.make_async_copy(
                outb.at[nb],
                out_hbm.at[pl.ds(0, _LPC), :, pl.ds(0, _G)],
                wsems[nb],
            ).wait()

    return gather_kernel


def kernel(input_ids, token_embedding):
    B, L = input_ids.shape
    V, D = token_embedding.shape
    tab2 = jnp.pad(token_embedding, ((0, 0), (0, 128 - D))).reshape(2 * V, D)
    idx_lm = (
        (input_ids.astype(jnp.int32) * 2)
        .reshape(_NW, _G, L)
        .transpose(0, 2, 1)
    )
    out3 = _build(B, L, D)(idx_lm, tab2)
    return out3.transpose(2, 0, 1)


# speedup vs baseline: 1.7935x; 1.7935x over previous
import functools

import jax
import jax.numpy as jnp
from jax import lax
from jax.experimental import pallas as pl
from jax.experimental.pallas import tpu as pltpu
from jax.experimental.pallas import tpu_sc as plsc

_NC = 2
_NS = 16
_NW = _NC * _NS
_G = 100
_GPC = 8
_CHUNK = _G * _GPC
_NBUF = 2


@functools.lru_cache(maxsize=None)
def _build(B, L, dim):
    total = B * L
    n_groups = total // _G
    groups_per_w = n_groups // _NW
    rows_per_w = total // _NW
    b_per_w = B // _NW
    chunks_per_w = rows_per_w // _CHUNK
    b_per_chunk = _CHUNK // L
    steps = chunks_per_w // _NBUF

    mesh = plsc.VectorSubcoreMesh(core_axis_name="c", subcore_axis_name="s")

    @functools.partial(
        pl.kernel,
        mesh=mesh,
        out_type=jax.ShapeDtypeStruct((B, L, dim), jnp.float32),
        compiler_params=pltpu.CompilerParams(use_tc_tiling_on_sc=False),
        scratch_types=[
            pltpu.VMEM((groups_per_w, _G), jnp.int32),
            pltpu.VMEM((_NBUF, b_per_chunk, L, dim), jnp.float32),
            pltpu.SemaphoreType.DMA,
            pltpu.SemaphoreType.DMA,
            pltpu.SemaphoreType.DMA,
            pltpu.SemaphoreType.DMA,
        ],
    )
    def gather_kernel(idx_hbm, table_hbm, out_hbm,
                      idx_v, rows, gsem0, gsem1, wsem0, wsem1):
        gsems = [gsem0, gsem1]
        wsems = [wsem0, wsem1]
        wid = lax.axis_index("s") * _NC + lax.axis_index("c")
        grp_base = wid * groups_per_w
        b_base = wid * b_per_w

        pltpu.sync_copy(idx_hbm.at[pl.ds(grp_base, groups_per_w)], idx_v)

        def step(i, carry):
            for b in range(_NBUF):
                c = i * _NBUF + b

                @pl.when(i >= 1)
                def _drain():
                    pltpu.make_async_copy(
                        rows.at[b],
                        out_hbm.at[pl.ds(0, b_per_chunk)],
                        wsems[b],
                    ).wait()

                copies = []
                g_per_l = L // _G
                for j in range(_GPC):
                    g = c * _GPC + j
                    copies.append(pltpu.make_async_copy(
                        table_hbm.at[idx_v.at[g]],
                        rows.at[b, j // g_per_l,
                                pl.ds((j % g_per_l) * _G, _G)],
                        gsems[b],
                    ))
                for cp in copies:
                    cp.start()
                for cp in copies:
                    cp.wait()

                pltpu.make_async_copy(
                    rows.at[b],
                    out_hbm.at[pl.ds(b_base + c * b_per_chunk, b_per_chunk)],
                    wsems[b],
                ).start()
            return carry

        lax.fori_loop(0, steps, step, 0)

        for b in range(_NBUF):
            pltpu.make_async_copy(
                rows.at[b], out_hbm.at[pl.ds(0, b_per_chunk)], wsems[b]
            ).wait()

    return gather_kernel


def kernel(input_ids, token_embedding):
    B, L = input_ids.shape
    V, D = token_embedding.shape
    total = B * L
    assert total % (_NW * _CHUNK * _NBUF) == 0 and _CHUNK % L == 0
    tab2 = jnp.pad(token_embedding, ((0, 0), (0, 128 - D))).reshape(2 * V, D)
    idx2 = (input_ids.astype(jnp.int32) * 2).reshape(total // _G, _G)
    return _build(B, L, D)(idx2, tab2)
